# baseline (speedup 1.0000x reference)
import functools

import jax
import jax.numpy as jnp
from jax import lax
from jax.experimental import pallas as pl
from jax.experimental.pallas import tpu as pltpu
from jax.experimental.pallas import tpu_sc as plsc

N_CLS = 100
CTX_DIM = 1024
EMBED_DIM = 512
BATCH = 16384

_NC, _NS = 2, 16
_NW = _NC * _NS
_B_PER_W = BATCH // _NW
_CHUNK = 64
_N_CHUNKS = _B_PER_W // _CHUNK
_LANES = 16
_VECS = EMBED_DIM // _LANES


def _table_body(base_ref, ctx_ref, w_ref, b_ref, out_ref):
    proj = jnp.dot(ctx_ref[...], w_ref[...], preferred_element_type=jnp.float32)
    feat = base_ref[...] + proj + b_ref[...]
    ss = jnp.sum(feat * feat, axis=1, keepdims=True)
    out_ref[...] = feat * lax.rsqrt(ss)


def _build_table(base, ctx, w, b2d):
    return pl.pallas_call(
        _table_body,
        out_shape=jax.ShapeDtypeStruct((N_CLS, EMBED_DIM), jnp.float32),
    )(base, ctx, w, b2d)


def _expand_body(table_hbm, idx_hbm, out_hbm, table_v, stage_v, idx_v,
                 ss0, ss1, st_sem):
    wid = lax.axis_index("s") * _NC + lax.axis_index("c")

    tab_cp = pltpu.async_copy(table_hbm, table_v, st_sem)
    pltpu.sync_copy(idx_hbm.at[pl.ds(wid * _B_PER_W, _B_PER_W)],
                    idx_v.at[pl.ds(0, _B_PER_W)])
    tab_cp.wait()

    ss = (ss0, ss1)
    scat = [None, None]

    def expand_chunk(j, buf):
        @plsc.parallel_loop(0, _CHUNK, 1, unroll=16)
        def _row(r):
            c = idx_v[pl.ds(j * _CHUNK + r, _LANES)][0]
            for k in range(_VECS):
                stage_v[buf, r, pl.ds(k * _LANES, _LANES)] = (
                    table_v[c, pl.ds(k * _LANES, _LANES)])

    for j in range(_N_CHUNKS):
        buf = j % 2
        if scat[buf] is not None:
            scat[buf].wait()
        expand_chunk(j, buf)
        row0 = wid * _B_PER_W + j * _CHUNK
        scat[buf] = pltpu.async_copy(
            stage_v.at[buf], out_hbm.at[pl.ds(row0, _CHUNK)], ss[buf])
    scat[0].wait()
    scat[1].wait()


@functools.lru_cache(maxsize=1)
def _make_expand():
    return pl.kernel(
        _expand_body,
        mesh=plsc.VectorSubcoreMesh(core_axis_name="c", subcore_axis_name="s"),
        out_type=jax.ShapeDtypeStruct((BATCH, EMBED_DIM), jnp.float32),
        scratch_types=[
            pltpu.VMEM((N_CLS, EMBED_DIM), jnp.float32),
            pltpu.VMEM((2, _CHUNK, EMBED_DIM), jnp.float32),
            pltpu.VMEM((_B_PER_W + _LANES,), jnp.int32),
            pltpu.SemaphoreType.DMA,
            pltpu.SemaphoreType.DMA,
            pltpu.SemaphoreType.DMA,
        ],
    )


def kernel(class_indices, base_features, prompt_ctx, W, b):
    table = _build_table(base_features, prompt_ctx, W, b.reshape(1, EMBED_DIM))
    return _make_expand()(table, class_indices)


# speedup vs baseline: 1.5548x; 1.5548x over previous
import functools

import jax
import jax.numpy as jnp
from jax import lax
from jax.experimental import pallas as pl
from jax.experimental.pallas import tpu as pltpu
from jax.experimental.pallas import tpu_sc as plsc

N_CLS = 100
CTX_DIM = 1024
EMBED_DIM = 512
BATCH = 16384

_NC, _NS = 2, 16
_NW = _NC * _NS
_B_PER_W = BATCH // _NW
_LANES = 16


def _table_body(base_ref, ctx_ref, w_ref, b_ref, out_ref):
    proj = jnp.dot(ctx_ref[...], w_ref[...], preferred_element_type=jnp.float32)
    feat = base_ref[...] + proj + b_ref[...]
    ss = jnp.sum(feat * feat, axis=1, keepdims=True)
    out_ref[...] = feat * lax.rsqrt(ss)


def _build_table(base, ctx, w, b2d):
    return pl.pallas_call(
        _table_body,
        out_shape=jax.ShapeDtypeStruct((N_CLS, EMBED_DIM), jnp.float32),
    )(base, ctx, w, b2d)


def _expand_body(table_hbm, idx_hbm, out_hbm, table_v, idx_v, ld_sem, row_sem):
    wid = lax.axis_index("s") * _NC + lax.axis_index("c")
    base = wid * _B_PER_W

    tab_cp = pltpu.async_copy(table_hbm, table_v, ld_sem)
    pltpu.sync_copy(idx_hbm.at[pl.ds(base, _B_PER_W)],
                    idx_v.at[pl.ds(0, _B_PER_W)])
    tab_cp.wait()

    @plsc.parallel_loop(0, _B_PER_W, 1, unroll=4)
    def _row(r):
        c = idx_v[pl.ds(r, _LANES)][0]
        pltpu.async_copy(table_v.at[c], out_hbm.at[base + r], row_sem)

    pltpu.make_async_copy(out_hbm.at[pl.ds(base, _B_PER_W)],
                          out_hbm.at[pl.ds(base, _B_PER_W)], row_sem).wait()


@functools.lru_cache(maxsize=1)
def _make_expand():
    return pl.kernel(
        _expand_body,
        mesh=plsc.VectorSubcoreMesh(core_axis_name="c", subcore_axis_name="s"),
        out_type=jax.ShapeDtypeStruct((BATCH, EMBED_DIM), jnp.float32),
        scratch_types=[
            pltpu.VMEM((N_CLS, EMBED_DIM), jnp.float32),
            pltpu.VMEM((_B_PER_W + _LANES,), jnp.int32),
            pltpu.SemaphoreType.DMA,
            pltpu.SemaphoreType.DMA,
        ],
    )


def kernel(class_indices, base_features, prompt_ctx, W, b):
    table = _build_table(base_features, prompt_ctx, W, b.reshape(1, EMBED_DIM))
    return _make_expand()(table, class_indices)
